# baseline (speedup 1.0000x reference)
import functools
import math

import jax
import jax.numpy as jnp
from jax.experimental import pallas as pl
from jax.experimental.pallas import tpu as pltpu

B = 32
N = 512
MEM = 16
D = 768
R = 4
TEMP = 0.7

NB = 32
T = B * NB
KM = NB * MEM
CB = 3072

_SQRT2 = math.sqrt(2.0)


def _gelu(v):
    return 0.5 * v * (1.0 + jax.lax.erf(v / _SQRT2))


def _mm(a, b):
    return jnp.dot(a.astype(jnp.bfloat16), b.astype(jnp.bfloat16),
                   preferred_element_type=jnp.float32)


def _mmt(a, b):
    return jax.lax.dot_general(
        a.astype(jnp.bfloat16), b.astype(jnp.bfloat16),
        (((1,), (1,)), ((), ())), preferred_element_type=jnp.float32)


def _adj_kernel(adj_ref, m_ref, o_ref):
    adj = adj_ref[...]
    r = jax.lax.broadcasted_iota(jnp.int32, (N, N), 0)
    c = jax.lax.broadcasted_iota(jnp.int32, (N, N), 1)
    a = adj + jnp.where(r == c, jnp.float32(1.0), jnp.float32(0.0))
    rowsum = jnp.sum(a, axis=1, keepdims=True)
    d = jnp.where(rowsum > 0, jax.lax.rsqrt(rowsum), jnp.float32(0.0))
    ab = (a * d).astype(jnp.bfloat16)
    m3 = (m_ref[...] * d[:, :, None]).astype(jnp.bfloat16)
    o = jax.lax.dot_general(ab, m3, (((1,), (0,)), ((), ())),
                            preferred_element_type=jnp.float32)
    o_ref[...] = o.astype(jnp.bfloat16)


def _bandshift(v, lidx, d):
    return jnp.where(lidx < MEM - d,
                     jnp.roll(v, -d, axis=1), jnp.roll(v, MEM - d, axis=1))


def _read(qn, keys, vals):
    k32 = keys.astype(jnp.float32)
    kn = k32 * jax.lax.rsqrt(
        jnp.maximum(jnp.sum(k32 * k32, axis=1, keepdims=True), 1e-24))
    sim_full = jax.lax.dot_general(
        qn.astype(jnp.bfloat16), kn.astype(jnp.bfloat16),
        (((1,), (1,)), ((), ())), preferred_element_type=jnp.float32)
    lane = jax.lax.broadcasted_iota(jnp.int32, (B, KM), 1)
    band = lane // MEM
    lidx = lane % MEM
    masks = [(band == n).astype(jnp.float32) for n in range(NB)]
    sc = masks[0] * sim_full[0:B, :]
    for n in range(1, NB):
        sc = sc + masks[n] * sim_full[n * B:(n + 1) * B, :]
    rank = jnp.zeros((B, KM), jnp.float32)
    for d in range(1, MEM):
        wrapped = lidx >= MEM - d
        nb = jnp.where(wrapped, jnp.roll(sc, MEM - d, axis=1),
                       jnp.roll(sc, -d, axis=1))
        rank = rank + jnp.where((nb > sc) | ((nb == sc) & wrapped),
                                jnp.float32(1.0), jnp.float32(0.0))
    sel = rank < R
    m = sc
    for d in (8, 4, 2, 1):
        m = jnp.maximum(m, _bandshift(m, lidx, d))
    e = jnp.where(sel, jnp.exp((sc - m) / TEMP), jnp.float32(0.0))
    sm = e
    for d in (8, 4, 2, 1):
        sm = sm + _bandshift(sm, lidx, d)
    w = e / sm
    wb = jnp.concatenate([w * masks[n] for n in range(NB)], axis=0)
    return _mm(wb, vals)


def _main_kernel(x_ref, mk_ref, mv_ref, nk_ref, nv_ref,
                 wq_ref, bq_ref, w1_ref, b1_ref,
                 w2_ref, b2_ref, wo_ref, bo_ref, lng_ref, lnb_ref, out_ref):
    xt = jnp.transpose(x_ref[...], (1, 0, 2)).reshape(T, D)
    xb = xt.astype(jnp.bfloat16)
    q = _mmt(xb, wq_ref[...]) + bq_ref[...]
    qn = q * jax.lax.rsqrt(
        jnp.maximum(jnp.sum(q * q, axis=1, keepdims=True), 1e-24))
    self_mem = _read(qn, mk_ref[...].reshape(KM, D), mv_ref[...].reshape(KM, D))
    nbr_mem = _read(qn, nk_ref[...].reshape(KM, D), nv_ref[...].reshape(KM, D))
    w1 = w1_ref[...]
    h = (_mmt(xb, w1[:, :D]) + _mmt(self_mem, w1[:, D:2 * D])
         + _mmt(nbr_mem, w1[:, 2 * D:]) + b1_ref[...])
    h = _gelu(h)
    fl = _mmt(h, w2_ref[...]) + b2_ref[...]
    fm = jnp.max(fl, axis=1, keepdims=True)
    fe = jnp.exp(fl - fm)
    fw = fe / jnp.sum(fe, axis=1, keepdims=True)
    fused = (fw[:, 0:1] * xt + fw[:, 1:2] * self_mem + fw[:, 2:3] * nbr_mem)
    o = _gelu(_mmt(fused, wo_ref[...]) + bo_ref[...])
    y = xt + o
    mu = jnp.mean(y, axis=1, keepdims=True)
    var = jnp.mean((y - mu) ** 2, axis=1, keepdims=True)
    yn = (y - mu) / jnp.sqrt(var + 1e-5) * lng_ref[...] + lnb_ref[...]
    out_ref[...] = jnp.transpose(yn.reshape(NB, B, D), (1, 0, 2))


def kernel(x, adj_mx, mem_keys, mem_vals, Wq, bq, W1, b1, W2, b2,
           Wo, bo, ln_g, ln_b):
    DB = D // 3
    m3_spec = pl.BlockSpec((N, MEM, DB), lambda e: (0, 0, e))
    adj_call = pl.pallas_call(
        _adj_kernel,
        grid=(D // DB,),
        in_specs=[
            pl.BlockSpec((N, N), lambda e: (0, 0)),
            m3_spec,
        ],
        out_specs=m3_spec,
        out_shape=jax.ShapeDtypeStruct((N, MEM, D), jnp.bfloat16),
    )
    nbr_k = adj_call(adj_mx, mem_keys)
    nbr_v = adj_call(adj_mx, mem_vals)

    full = lambda n: (0, 0)
    mem_spec = pl.BlockSpec((NB, MEM, D), lambda n: (n, 0, 0))
    out = pl.pallas_call(
        _main_kernel,
        grid=(N // NB,),
        in_specs=[
            pl.BlockSpec((B, NB, D), lambda n: (0, n, 0)),
            mem_spec, mem_spec, mem_spec, mem_spec,
            pl.BlockSpec((D, D), full),
            pl.BlockSpec((1, D), full),
            pl.BlockSpec((D, 3 * D), full),
            pl.BlockSpec((1, D), full),
            pl.BlockSpec((3, D), full),
            pl.BlockSpec((1, 3), full),
            pl.BlockSpec((D, D), full),
            pl.BlockSpec((1, D), full),
            pl.BlockSpec((1, D), full),
            pl.BlockSpec((1, D), full),
        ],
        out_specs=pl.BlockSpec((B, NB, D), lambda n: (0, n, 0)),
        out_shape=jax.ShapeDtypeStruct((B, N, D), jnp.float32),
    )(
        x, mem_keys, mem_vals, nbr_k, nbr_v,
        Wq.astype(jnp.bfloat16), bq.reshape(1, D),
        W1.astype(jnp.bfloat16), b1.reshape(1, D),
        W2.astype(jnp.bfloat16), b2.reshape(1, 3),
        Wo.astype(jnp.bfloat16), bo.reshape(1, D),
        ln_g.reshape(1, D), ln_b.reshape(1, D),
    )
    return out


# speedup vs baseline: 1.0349x; 1.0349x over previous
import functools
import math

import jax
import jax.numpy as jnp
from jax.experimental import pallas as pl
from jax.experimental.pallas import tpu as pltpu

B = 32
N = 512
MEM = 16
D = 768
R = 4
TEMP = 0.7

NB = 16
T = B * NB
KM = NB * MEM
CB = 3072

_SQRT2 = math.sqrt(2.0)


def _gelu(v):
    return 0.5 * v * (1.0 + jax.lax.erf(v / _SQRT2))


def _mm(a, b):
    return jnp.dot(a.astype(jnp.bfloat16), b.astype(jnp.bfloat16),
                   preferred_element_type=jnp.float32)


def _mmt(a, b):
    return jax.lax.dot_general(
        a.astype(jnp.bfloat16), b.astype(jnp.bfloat16),
        (((1,), (1,)), ((), ())), preferred_element_type=jnp.float32)


def _adj_kernel(adj_ref, m_ref, o_ref):
    adj = adj_ref[...]
    r = jax.lax.broadcasted_iota(jnp.int32, (N, N), 0)
    c = jax.lax.broadcasted_iota(jnp.int32, (N, N), 1)
    a = adj + jnp.where(r == c, jnp.float32(1.0), jnp.float32(0.0))
    rowsum = jnp.sum(a, axis=1, keepdims=True)
    d = jnp.where(rowsum > 0, jax.lax.rsqrt(rowsum), jnp.float32(0.0))
    ab = (a * d).astype(jnp.bfloat16)
    m3 = (m_ref[...] * d[:, :, None]).astype(jnp.bfloat16)
    o = jax.lax.dot_general(ab, m3, (((1,), (0,)), ((), ())),
                            preferred_element_type=jnp.float32)
    o_ref[...] = o.astype(jnp.bfloat16)


def _bandshift(v, lidx, d):
    return jnp.where(lidx < MEM - d,
                     jnp.roll(v, -d, axis=1), jnp.roll(v, MEM - d, axis=1))


def _read(qn, keys, vals):
    k32 = keys.astype(jnp.float32)
    kn = k32 * jax.lax.rsqrt(
        jnp.maximum(jnp.sum(k32 * k32, axis=1, keepdims=True), 1e-24))
    sim_full = jax.lax.dot_general(
        qn.astype(jnp.bfloat16), kn.astype(jnp.bfloat16),
        (((1,), (1,)), ((), ())), preferred_element_type=jnp.float32)
    lane = jax.lax.broadcasted_iota(jnp.int32, (B, KM), 1)
    band = lane // MEM
    lidx = lane % MEM
    masks = [(band == n).astype(jnp.float32) for n in range(NB)]
    sc = masks[0] * sim_full[0:B, :]
    for n in range(1, NB):
        sc = sc + masks[n] * sim_full[n * B:(n + 1) * B, :]
    rank = jnp.zeros((B, KM), jnp.float32)
    for d in range(1, MEM):
        wrapped = lidx >= MEM - d
        nb = jnp.where(wrapped, jnp.roll(sc, MEM - d, axis=1),
                       jnp.roll(sc, -d, axis=1))
        rank = rank + jnp.where((nb > sc) | ((nb == sc) & wrapped),
                                jnp.float32(1.0), jnp.float32(0.0))
    sel = rank < R
    m = sc
    for d in (8, 4, 2, 1):
        m = jnp.maximum(m, _bandshift(m, lidx, d))
    e = jnp.where(sel, jnp.exp((sc - m) / TEMP), jnp.float32(0.0))
    sm = e
    for d in (8, 4, 2, 1):
        sm = sm + _bandshift(sm, lidx, d)
    w = e / sm
    wb = jnp.concatenate([w * masks[n] for n in range(NB)], axis=0)
    return _mm(wb, vals)


def _main_kernel(x_ref, mk_ref, mv_ref, nk_ref, nv_ref,
                 wq_ref, bq_ref, w1_ref, b1_ref,
                 w2_ref, b2_ref, wo_ref, bo_ref, lng_ref, lnb_ref, out_ref):
    xt = jnp.transpose(x_ref[...], (1, 0, 2)).reshape(T, D)
    xb = xt.astype(jnp.bfloat16)
    q = _mmt(xb, wq_ref[...]) + bq_ref[...]
    qn = q * jax.lax.rsqrt(
        jnp.maximum(jnp.sum(q * q, axis=1, keepdims=True), 1e-24))
    self_mem = _read(qn, mk_ref[...].reshape(KM, D), mv_ref[...].reshape(KM, D))
    nbr_mem = _read(qn, nk_ref[...].reshape(KM, D), nv_ref[...].reshape(KM, D))
    w1 = w1_ref[...]
    h = (_mmt(xb, w1[:, :D]) + _mmt(self_mem, w1[:, D:2 * D])
         + _mmt(nbr_mem, w1[:, 2 * D:]) + b1_ref[...])
    h = _gelu(h)
    fl = _mmt(h, w2_ref[...]) + b2_ref[...]
    fm = jnp.max(fl, axis=1, keepdims=True)
    fe = jnp.exp(fl - fm)
    fw = fe / jnp.sum(fe, axis=1, keepdims=True)
    fused = (fw[:, 0:1] * xt + fw[:, 1:2] * self_mem + fw[:, 2:3] * nbr_mem)
    o = _gelu(_mmt(fused, wo_ref[...]) + bo_ref[...])
    y = xt + o
    mu = jnp.mean(y, axis=1, keepdims=True)
    var = jnp.mean((y - mu) ** 2, axis=1, keepdims=True)
    yn = (y - mu) / jnp.sqrt(var + 1e-5) * lng_ref[...] + lnb_ref[...]
    out_ref[...] = jnp.transpose(yn.reshape(NB, B, D), (1, 0, 2))


def kernel(x, adj_mx, mem_keys, mem_vals, Wq, bq, W1, b1, W2, b2,
           Wo, bo, ln_g, ln_b):
    DB = D // 3
    m3_spec = pl.BlockSpec((N, MEM, DB), lambda e: (0, 0, e))
    adj_call = pl.pallas_call(
        _adj_kernel,
        grid=(D // DB,),
        in_specs=[
            pl.BlockSpec((N, N), lambda e: (0, 0)),
            m3_spec,
        ],
        out_specs=m3_spec,
        out_shape=jax.ShapeDtypeStruct((N, MEM, D), jnp.bfloat16),
    )
    nbr_k = adj_call(adj_mx, mem_keys)
    nbr_v = adj_call(adj_mx, mem_vals)

    full = lambda n: (0, 0)
    mem_spec = pl.BlockSpec((NB, MEM, D), lambda n: (n, 0, 0))
    out = pl.pallas_call(
        _main_kernel,
        grid=(N // NB,),
        in_specs=[
            pl.BlockSpec((B, NB, D), lambda n: (0, n, 0)),
            mem_spec, mem_spec, mem_spec, mem_spec,
            pl.BlockSpec((D, D), full),
            pl.BlockSpec((1, D), full),
            pl.BlockSpec((D, 3 * D), full),
            pl.BlockSpec((1, D), full),
            pl.BlockSpec((3, D), full),
            pl.BlockSpec((1, 3), full),
            pl.BlockSpec((D, D), full),
            pl.BlockSpec((1, D), full),
            pl.BlockSpec((1, D), full),
            pl.BlockSpec((1, D), full),
        ],
        out_specs=pl.BlockSpec((B, NB, D), lambda n: (0, n, 0)),
        out_shape=jax.ShapeDtypeStruct((B, N, D), jnp.float32),
    )(
        x, mem_keys, mem_vals, nbr_k, nbr_v,
        Wq.astype(jnp.bfloat16), bq.reshape(1, D),
        W1.astype(jnp.bfloat16), b1.reshape(1, D),
        W2.astype(jnp.bfloat16), b2.reshape(1, 3),
        Wo.astype(jnp.bfloat16), bo.reshape(1, D),
        ln_g.reshape(1, D), ln_b.reshape(1, D),
    )
    return out
